# baseline (speedup 1.0000x reference)
import functools

import jax
import jax.numpy as jnp
from jax import lax
from jax.experimental import pallas as pl
from jax.experimental.pallas import tpu as pltpu
from jax.experimental.pallas import tpu_sc as plsc

_NC = 2
_NS = 16
_NW = _NC * _NS
_CH = 128
_NSL = 25
_L = 16


def _table_body(charge_ref, temp_ref, emb_ref, w1_ref, w2_ref, wpa_ref,
                wpb_ref, out_ref):
    b = charge_ref.shape[0]
    vp = emb_ref.shape[0]
    charge = charge_ref[...]
    iota = lax.broadcasted_iota(jnp.int32, (b, vp), 1)
    oh = jnp.where(charge == iota, 1.0, 0.0).astype(jnp.float32)
    e_charge = jnp.dot(oh, emb_ref[...],
                       preferred_element_type=jnp.float32)
    t = temp_ref[...]
    h = t * w1_ref[...]
    h = h * jax.nn.sigmoid(h)
    e_temp = jnp.dot(h, w2_ref[...],
                     preferred_element_type=jnp.float32)
    y = (jnp.dot(e_charge, wpa_ref[...], preferred_element_type=jnp.float32)
         + jnp.dot(e_temp, wpb_ref[...], preferred_element_type=jnp.float32))
    out_ref[...] = y * jax.nn.sigmoid(y)


def _make_expand(n, b, d):
    atoms_per_w = _NSL * _CH
    n_up = ((n + 127) // 128) * 128
    mesh = plsc.VectorSubcoreMesh(core_axis_name="c", subcore_axis_name="s")

    @functools.partial(
        pl.kernel,
        mesh=mesh,
        out_type=jax.ShapeDtypeStruct((d, n), jnp.float32),
        scratch_types=[
            pltpu.VMEM((b // 2, 2 * d), jnp.float32),
            pltpu.VMEM((atoms_per_w,), jnp.int32),
            pltpu.VMEM((d, _CH), jnp.float32),
            pltpu.VMEM((d, _CH), jnp.float32),
            pltpu.SemaphoreType.DMA,
            pltpu.SemaphoreType.DMA,
        ],
        compiler_params=pltpu.CompilerParams(use_tc_tiling_on_sc=True,
                                             needs_layout_passes=False,
                                             disable_bounds_checks=True),
    )
    def expand_k(table_hbm, batch_hbm, out_hbm, table_v, idx_v,
                 sl0, sl1, ss0, ss1):
        slabs = (sl0, sl1)
        ss = (ss0, ss1)
        wid = lax.axis_index("s") * _NC + lax.axis_index("c")
        woff = jnp.minimum(wid * atoms_per_w, n_up - atoms_per_w)
        pltpu.sync_copy(table_hbm, table_v)
        pltpu.sync_copy(batch_hbm.at[pl.ds(woff, atoms_per_w)], idx_v)

        def store(sl, p):
            pltpu.async_copy(
                slabs[p], out_hbm.at[:, pl.ds(woff + sl * _CH, _CH)], ss[p])

        def wait_store(sl, p):
            pltpu.make_async_copy(
                slabs[p], out_hbm.at[:, pl.ds(woff + sl * _CH, _CH)],
                ss[p]).wait()

        def do_slab(sl, p):
            slab = slabs[p]

            @plsc.parallel_loop(0, _CH, step=_L, unroll=2)
            def _group(i):
                raw = idx_v[pl.ds(sl * _CH + i, _L)]
                idx16 = jnp.clip(raw, 0, b - 1)
                row16 = lax.shift_right_logical(idx16, 1)
                colb = lax.shift_left(jnp.bitwise_and(idx16, 1), 6)
                for f0 in range(0, d, 4):
                    vs = [plsc.load_gather(table_v, [row16, colb + (f0 + k)])
                          for k in range(4)]
                    for k in range(4):
                        slab[f0 + k, pl.ds(i, _L)] = vs[k]

            store(sl, p)

        do_slab(0, 0)

        def pair(pp, carry):
            for half in range(2):
                sl_ = pp * 2 + 1 + half
                p = 1 - half

                @pl.when(sl_ >= 2)
                def _():
                    wait_store(sl_ - 2, p)

                do_slab(sl_, p)
            return carry

        lax.fori_loop(0, (_NSL - 1) // 2, pair, 0, unroll=False)
        wait_store(_NSL - 2, (_NSL - 2) % 2)
        wait_store(_NSL - 1, (_NSL - 1) % 2)

    return expand_k


def kernel(batch, charge, temperature, emb_charge, W1, W2, Wp):
    n = batch.shape[0]
    b = charge.shape[0]
    v, d = emb_charge.shape

    vp = ((v + 127) // 128) * 128
    emb_pad = jnp.zeros((vp, d), jnp.float32).at[:v].set(emb_charge)
    table = pl.pallas_call(
        _table_body,
        out_shape=jax.ShapeDtypeStruct((b, d), jnp.float32),
    )(charge.astype(jnp.int32).reshape(b, 1),
      temperature.reshape(b, 1),
      emb_pad, W1, W2, Wp[:d, :], Wp[d:, :])

    assert _NW * _NSL * _CH >= ((n + 127) // 128) * 128
    assert _NSL * _CH <= n and _CH % 128 == 0
    table_pairs = table.reshape(b // 2, 2 * d)
    out_t = _make_expand(n, b, d)(table_pairs, batch.astype(jnp.int32))
    return out_t.T


# speedup vs baseline: 1.0331x; 1.0331x over previous
import functools

import jax
import jax.numpy as jnp
from jax import lax
from jax.experimental import pallas as pl
from jax.experimental.pallas import tpu as pltpu
from jax.experimental.pallas import tpu_sc as plsc

_NC = 2
_NS = 16
_NW = _NC * _NS
_CH = 256
_NSL = 13
_L = 16


def _table_body(charge_ref, temp_ref, emb_ref, w1_ref, w2_ref, wpa_ref,
                wpb_ref, out_ref):
    b = charge_ref.shape[0]
    vp = emb_ref.shape[0]
    charge = charge_ref[...]
    iota = lax.broadcasted_iota(jnp.int32, (b, vp), 1)
    oh = jnp.where(charge == iota, 1.0, 0.0).astype(jnp.float32)
    e_charge = jnp.dot(oh, emb_ref[...],
                       preferred_element_type=jnp.float32)
    t = temp_ref[...]
    h = t * w1_ref[...]
    h = h * jax.nn.sigmoid(h)
    e_temp = jnp.dot(h, w2_ref[...],
                     preferred_element_type=jnp.float32)
    y = (jnp.dot(e_charge, wpa_ref[...], preferred_element_type=jnp.float32)
         + jnp.dot(e_temp, wpb_ref[...], preferred_element_type=jnp.float32))
    out_ref[...] = y * jax.nn.sigmoid(y)


def _make_expand(n, b, d):
    atoms_per_w = _NSL * _CH
    n_up = ((n + 127) // 128) * 128
    mesh = plsc.VectorSubcoreMesh(core_axis_name="c", subcore_axis_name="s")

    @functools.partial(
        pl.kernel,
        mesh=mesh,
        out_type=jax.ShapeDtypeStruct((d, n), jnp.float32),
        scratch_types=[
            pltpu.VMEM((b // 2, 2 * d), jnp.float32),
            pltpu.VMEM((atoms_per_w,), jnp.int32),
            pltpu.VMEM((d, _CH), jnp.float32),
            pltpu.VMEM((d, _CH), jnp.float32),
            pltpu.SemaphoreType.DMA,
            pltpu.SemaphoreType.DMA,
        ],
        compiler_params=pltpu.CompilerParams(use_tc_tiling_on_sc=True,
                                             needs_layout_passes=False,
                                             disable_bounds_checks=True),
    )
    def expand_k(table_hbm, batch_hbm, out_hbm, table_v, idx_v,
                 sl0, sl1, ss0, ss1):
        slabs = (sl0, sl1)
        ss = (ss0, ss1)
        wid = lax.axis_index("s") * _NC + lax.axis_index("c")
        woff = jnp.minimum(wid * atoms_per_w, n_up - atoms_per_w)
        pltpu.sync_copy(table_hbm, table_v)
        pltpu.sync_copy(batch_hbm.at[pl.ds(woff, atoms_per_w)], idx_v)

        def store(sl, p):
            pltpu.async_copy(
                slabs[p], out_hbm.at[:, pl.ds(woff + sl * _CH, _CH)], ss[p])

        def wait_store(sl, p):
            pltpu.make_async_copy(
                slabs[p], out_hbm.at[:, pl.ds(woff + sl * _CH, _CH)],
                ss[p]).wait()

        def do_slab(sl, p):
            slab = slabs[p]

            @plsc.parallel_loop(0, _CH, step=_L, unroll=1)
            def _group(i):
                raw = idx_v[pl.ds(sl * _CH + i, _L)]
                idx16 = jnp.clip(raw, 0, b - 1)
                row16 = lax.shift_right_logical(idx16, 1)
                colb = lax.shift_left(jnp.bitwise_and(idx16, 1), 6)
                for f0 in range(0, d, 4):
                    vs = [plsc.load_gather(table_v, [row16, colb + (f0 + k)])
                          for k in range(4)]
                    for k in range(4):
                        slab[f0 + k, pl.ds(i, _L)] = vs[k]

            store(sl, p)

        do_slab(0, 0)

        def pair(pp, carry):
            for half in range(2):
                sl_ = pp * 2 + 1 + half
                p = 1 - half

                @pl.when(sl_ >= 2)
                def _():
                    wait_store(sl_ - 2, p)

                do_slab(sl_, p)
            return carry

        lax.fori_loop(0, (_NSL - 1) // 2, pair, 0, unroll=False)
        wait_store(_NSL - 2, (_NSL - 2) % 2)
        wait_store(_NSL - 1, (_NSL - 1) % 2)

    return expand_k


def kernel(batch, charge, temperature, emb_charge, W1, W2, Wp):
    n = batch.shape[0]
    b = charge.shape[0]
    v, d = emb_charge.shape

    vp = ((v + 127) // 128) * 128
    emb_pad = jnp.zeros((vp, d), jnp.float32).at[:v].set(emb_charge)
    table = pl.pallas_call(
        _table_body,
        out_shape=jax.ShapeDtypeStruct((b, d), jnp.float32),
    )(charge.astype(jnp.int32).reshape(b, 1),
      temperature.reshape(b, 1),
      emb_pad, W1, W2, Wp[:d, :], Wp[d:, :])

    assert _NW * _NSL * _CH >= ((n + 127) // 128) * 128
    assert _NSL * _CH <= n and _CH % 128 == 0
    table_pairs = table.reshape(b // 2, 2 * d)
    out_t = _make_expand(n, b, d)(table_pairs, batch.astype(jnp.int32))
    return out_t.T


# speedup vs baseline: 1.0642x; 1.0301x over previous
import functools

import jax
import jax.numpy as jnp
from jax import lax
from jax.experimental import pallas as pl
from jax.experimental.pallas import tpu as pltpu
from jax.experimental.pallas import tpu_sc as plsc

_NC = 2
_NS = 16
_NW = _NC * _NS
_CH = 128
_NSL = 25
_L = 16


def _table_body(charge_ref, temp_ref, emb_ref, w1_ref, w2_ref, wp_ref,
                out_ref):
    b = charge_ref.shape[0]
    vp = emb_ref.shape[0]
    charge = charge_ref[...]
    iota = lax.broadcasted_iota(jnp.int32, (b, vp), 1)
    oh = jnp.where(charge == iota, 1.0, 0.0).astype(jnp.float32)
    e_charge = jnp.dot(oh, emb_ref[...],
                       preferred_element_type=jnp.float32)
    t = temp_ref[...]
    h = t * w1_ref[...]
    h = h * jax.nn.sigmoid(h)
    e_temp = jnp.dot(h, w2_ref[...],
                     preferred_element_type=jnp.float32)
    d = e_charge.shape[1]
    y = (jnp.dot(e_charge, wp_ref[:d, :], preferred_element_type=jnp.float32)
         + jnp.dot(e_temp, wp_ref[d:, :], preferred_element_type=jnp.float32))
    out_ref[...] = y * jax.nn.sigmoid(y)


def _make_expand(n, b, d):
    atoms_per_w = _NSL * _CH
    n_up = ((n + 127) // 128) * 128
    mesh = plsc.VectorSubcoreMesh(core_axis_name="c", subcore_axis_name="s")

    @functools.partial(
        pl.kernel,
        mesh=mesh,
        out_type=jax.ShapeDtypeStruct((d, n), jnp.float32),
        scratch_types=[
            pltpu.VMEM((b // 2, 2 * d), jnp.float32),
            pltpu.VMEM((atoms_per_w,), jnp.int32),
            pltpu.VMEM((d, _CH), jnp.float32),
            pltpu.VMEM((d, _CH), jnp.float32),
            pltpu.SemaphoreType.DMA,
            pltpu.SemaphoreType.DMA,
        ],
        compiler_params=pltpu.CompilerParams(use_tc_tiling_on_sc=True,
                                             needs_layout_passes=False,
                                             disable_bounds_checks=True),
    )
    def expand_k(table_hbm, batch_hbm, out_hbm, table_v, idx_v,
                 sl0, sl1, ss0, ss1):
        slabs = (sl0, sl1)
        ss = (ss0, ss1)
        wid = lax.axis_index("s") * _NC + lax.axis_index("c")
        woff = jnp.minimum(wid * atoms_per_w, n_up - atoms_per_w)
        pltpu.sync_copy(table_hbm, table_v)
        pltpu.sync_copy(batch_hbm.at[pl.ds(woff, atoms_per_w)], idx_v)

        def store(sl, p):
            pltpu.async_copy(
                slabs[p], out_hbm.at[:, pl.ds(woff + sl * _CH, _CH)], ss[p])

        def wait_store(sl, p):
            pltpu.make_async_copy(
                slabs[p], out_hbm.at[:, pl.ds(woff + sl * _CH, _CH)],
                ss[p]).wait()

        def do_slab(sl, p):
            slab = slabs[p]

            @plsc.parallel_loop(0, _CH, step=_L, unroll=1)
            def _group(i):
                raw = idx_v[pl.ds(sl * _CH + i, _L)]
                idx16 = jnp.clip(raw, 0, b - 1)
                row16 = lax.shift_right_logical(idx16, 1)
                colb = lax.shift_left(jnp.bitwise_and(idx16, 1), 6)
                for f0 in range(0, d, 4):
                    vs = [plsc.load_gather(table_v, [row16, colb + (f0 + k)])
                          for k in range(4)]
                    for k in range(4):
                        slab[f0 + k, pl.ds(i, _L)] = vs[k]

            store(sl, p)

        do_slab(0, 0)

        def pair(pp, carry):
            for half in range(2):
                sl_ = pp * 2 + 1 + half
                p = 1 - half

                @pl.when(sl_ >= 2)
                def _():
                    wait_store(sl_ - 2, p)

                do_slab(sl_, p)
            return carry

        lax.fori_loop(0, (_NSL - 1) // 2, pair, 0, unroll=False)
        wait_store(_NSL - 2, (_NSL - 2) % 2)
        wait_store(_NSL - 1, (_NSL - 1) % 2)

    return expand_k


def kernel(batch, charge, temperature, emb_charge, W1, W2, Wp):
    n = batch.shape[0]
    b = charge.shape[0]
    v, d = emb_charge.shape

    table = pl.pallas_call(
        _table_body,
        out_shape=jax.ShapeDtypeStruct((b, d), jnp.float32),
    )(charge.astype(jnp.int32).reshape(b, 1),
      temperature.reshape(b, 1),
      emb_charge, W1, W2, Wp)
    table_pairs = table.reshape(b // 2, 2 * d)

    assert _NW * _NSL * _CH >= ((n + 127) // 128) * 128
    assert _NSL * _CH <= n and _CH % 128 == 0
    out_t = _make_expand(n, b, d)(table_pairs, batch.astype(jnp.int32))
    return out_t.T
